# baseline (speedup 1.0000x reference)
import jax
import jax.numpy as jnp
from jax import lax
from jax.experimental import pallas as pl
from jax.experimental.pallas import tpu as pltpu
from jax.experimental.pallas import tpu_sc as plsc

RES = 128
N = RES ** 3
NC, NS = 2, 16
NW = NC * NS
C = N // NW
S = 4096
GROUPS = S // 16
BATCHES = S // 128
SUBCH = C // S
CNT_S = 8192

_CPARAMS = pltpu.CompilerParams(needs_layout_passes=False, disable_bounds_checks=True)


def _wid():
    return lax.axis_index("s") * NC + lax.axis_index("c")


def _count_body(mask_hbm, counts_hbm, mbuf, cbuf):
    wid = _wid()
    acc = jnp.zeros((16,), jnp.int32)
    for si in range(C // CNT_S):
        base = pl.multiple_of(wid * C + si * CNT_S, CNT_S)
        pltpu.sync_copy(mask_hbm.at[pl.ds(base, CNT_S)], mbuf)
        acc = lax.fori_loop(
            0, CNT_S // 16,
            lambda g, a: a + mbuf[pl.ds(g * 16, 16)],
            acc)
    cbuf[...] = jnp.full((16,), jnp.sum(acc), jnp.int32)
    pltpu.sync_copy(cbuf, counts_hbm.at[pl.ds(pl.multiple_of(wid * 16, 16), 16)])


def _main_body(mask_hbm, noise_hbm, params_hbm, counts_hbm,
               xw_hbm, occ_hbm,
               mbuf, nbuf, ysx, ysy, ysz, ostag, istag, pbuf, cntbuf, offbuf,
               sem_y, sem_o):
    wid = _wid()
    lanes = lax.iota(jnp.int32, 16)

    pltpu.sync_copy(params_hbm, pbuf)
    pltpu.sync_copy(counts_hbm, cntbuf)

    g0 = plsc.load_gather(cntbuf, [lanes * 16])
    g1 = plsc.load_gather(cntbuf, [256 + lanes * 16])
    tot0 = jnp.sum(g0)
    total = tot0 + jnp.sum(g1)
    e0 = plsc.cumsum(g0) - g0
    e1 = plsc.cumsum(g1) - g1 + tot0
    offbuf[pl.ds(0, 16)] = e0
    offbuf[pl.ds(16, 16)] = e1
    widv = jnp.full((16,), wid, jnp.int32)
    off1v = plsc.load_gather(offbuf, [widv])
    off0v = (total + wid * C) - off1v

    sxv, syv, szv = pbuf[0, :], pbuf[1, :], pbuf[2, :]
    oxv, oyv, ozv = pbuf[3, :], pbuf[4, :], pbuf[5, :]
    yfx, yfy, yfz = pbuf[6, :], pbuf[7, :], pbuf[8, :]

    chunk0 = wid * C

    def subchunk(si, carry):
        off1v, off0v = carry
        base = pl.multiple_of(chunk0 + si * S, S)
        pltpu.sync_copy(mask_hbm.at[pl.ds(base, S)], mbuf)
        pltpu.sync_copy(noise_hbm.at[pl.ds(base * 3, 3 * S)], nbuf)

        def group(g, c):
            o1, o0 = c
            mv = mbuf[pl.ds(g * 16, 16)]
            m = mv > 0
            incl = plsc.cumsum(mv)
            excl = incl - mv
            t1 = jnp.sum(mv)
            dst = jnp.where(m, o1 + excl, o0 + (lanes - excl))
            lrow = g * 16 + lanes
            vidx = base + lrow
            occv = jnp.where(m, vidx, 0)
            fi = (vidx >> 14).astype(jnp.float32)
            fj = ((vidx >> 7) & 127).astype(jnp.float32)
            fk = (vidx & 127).astype(jnp.float32)
            nx = plsc.load_gather(nbuf, [lrow * 3])
            ny = plsc.load_gather(nbuf, [lrow * 3 + 1])
            nz = plsc.load_gather(nbuf, [lrow * 3 + 2])
            yx = jnp.where(m, (fi + nx) * sxv + oxv, yfx)
            yy = jnp.where(m, (fj + ny) * syv + oyv, yfy)
            yz = jnp.where(m, (fk + nz) * szv + ozv, yfz)
            pos = g * 16
            ysx[pl.ds(pos, 16)] = yx
            ysy[pl.ds(pos, 16)] = yy
            ysz[pl.ds(pos, 16)] = yz
            ostag[pl.ds(pos, 16)] = occv
            row0 = jnp.full((16,), (g >> 3) * 4, jnp.int32)
            col = (g & 7) * 16 + lanes
            dst3 = dst * 3
            plsc.store_scatter(istag, [row0, col], dst)
            plsc.store_scatter(istag, [row0 + 1, col], dst3)
            plsc.store_scatter(istag, [row0 + 2, col], dst3 + 1)
            plsc.store_scatter(istag, [row0 + 3, col], dst3 + 2)
            return (o1 + t1, o0 + (16 - t1))

        off1v, off0v = lax.fori_loop(0, GROUPS, group, (off1v, off0v))

        copies = []
        for b in range(BATCHES):
            sl = pl.ds(b * 128, 128)
            copies.append(
                pltpu.async_copy(ostag.at[sl], occ_hbm.at[istag.at[4 * b]],
                                 sem_o))
            copies.append(
                pltpu.async_copy(ysx.at[sl], xw_hbm.at[istag.at[4 * b + 1]],
                                 sem_y))
            copies.append(
                pltpu.async_copy(ysy.at[sl], xw_hbm.at[istag.at[4 * b + 2]],
                                 sem_y))
            copies.append(
                pltpu.async_copy(ysz.at[sl], xw_hbm.at[istag.at[4 * b + 3]],
                                 sem_y))
        for cp in copies:
            cp.wait()
        return (off1v, off0v)

    lax.fori_loop(0, SUBCH, subchunk, (off1v, off0v))


def kernel(binary, noise, roi_aabb):
    mask = binary.reshape(-1).astype(jnp.int32)
    noise_flat = noise.reshape(-1)
    scale = (roi_aabb[3:] - roi_aabb[:3]) / jnp.float32(RES)
    offset = roi_aabb[:3]
    yfill = noise[0] * scale + offset
    params = jnp.tile(
        jnp.concatenate([scale, offset, yfill])[:, None], (1, 16))

    mesh = plsc.VectorSubcoreMesh(
        core_axis_name="c", subcore_axis_name="s",
        num_cores=NC, num_subcores=NS)

    count_k = pl.kernel(
        _count_body,
        out_type=jax.ShapeDtypeStruct((NW * 16,), jnp.int32),
        mesh=mesh,
        compiler_params=_CPARAMS,
        scratch_types=[
            pltpu.VMEM((CNT_S,), jnp.int32),
            pltpu.VMEM((16,), jnp.int32),
        ])
    counts = count_k(mask)

    main_k = pl.kernel(
        _main_body,
        out_type=(jax.ShapeDtypeStruct((3 * N,), jnp.float32),
                  jax.ShapeDtypeStruct((N,), jnp.int32)),
        mesh=mesh,
        compiler_params=_CPARAMS,
        scratch_types=[
            pltpu.VMEM((S,), jnp.int32),
            pltpu.VMEM((3 * S,), jnp.float32),
            pltpu.VMEM((S,), jnp.float32),
            pltpu.VMEM((S,), jnp.float32),
            pltpu.VMEM((S,), jnp.float32),
            pltpu.VMEM((S,), jnp.int32),
            pltpu.VMEM((4 * BATCHES, 128), jnp.int32),
            pltpu.VMEM((9, 16), jnp.float32),
            pltpu.VMEM((NW * 16,), jnp.int32),
            pltpu.VMEM((32,), jnp.int32),
            pltpu.SemaphoreType.DMA,
            pltpu.SemaphoreType.DMA,
        ])
    xw_flat, occ_indices = main_k(mask, noise_flat, params, counts)
    return xw_flat.reshape(N, 3), occ_indices


# speedup vs baseline: 4.5419x; 4.5419x over previous
import jax
import jax.numpy as jnp
from jax import lax
from jax.experimental import pallas as pl
from jax.experimental.pallas import tpu as pltpu
from jax.experimental.pallas import tpu_sc as plsc

RES = 128
N = RES ** 3
NC, NS = 2, 16
NW = NC * NS
C = N // NW
S = 4096
GROUPS = S // 16
SUBCH = C // S
CNT_S = 8192
RING = 8192
RM = RING - 1
BLK = 2048

_CPARAMS = pltpu.CompilerParams(
    needs_layout_passes=False, disable_bounds_checks=True)


def _mo8(x):
    return pl.multiple_of(x, 8)


def _wid():
    return lax.axis_index("s") * NC + lax.axis_index("c")


def _count_body(mask_hbm, counts_hbm, mbuf, cbuf):
    wid = _wid()
    acc = jnp.zeros((16,), jnp.int32)
    for si in range(C // CNT_S):
        base = pl.multiple_of(wid * C + si * CNT_S, CNT_S)
        pltpu.sync_copy(mask_hbm.at[pl.ds(base, CNT_S)], mbuf)
        acc = lax.fori_loop(
            0, CNT_S // 16,
            lambda g, a: a + mbuf[pl.ds(g * 16, 16)],
            acc)
    cbuf[...] = jnp.full((16,), jnp.sum(acc), jnp.int32)
    pltpu.sync_copy(cbuf, counts_hbm.at[pl.ds(pl.multiple_of(wid * 16, 16), 16)])


def _main_body(mask_hbm, noise_hbm, params_hbm, counts_hbm, fpat_hbm, zpat_hbm,
               xw_hbm, occ_hbm,
               mbuf, nbuf, oring, xring, pbuf, cntbuf, offbuf,
               zbufv, fbufv, eidx, evali, evalf,
               sem_e, sem_f):
    wid = _wid()
    lanes = lax.iota(jnp.int32, 16)

    pltpu.sync_copy(params_hbm, pbuf)
    pltpu.sync_copy(counts_hbm, cntbuf)
    pltpu.sync_copy(zpat_hbm, zbufv)
    pltpu.sync_copy(fpat_hbm, fbufv)

    g0 = plsc.load_gather(cntbuf, [lanes * 16])
    g1 = plsc.load_gather(cntbuf, [256 + lanes * 16])
    tot0 = jnp.sum(g0)
    total = tot0 + jnp.sum(g1)
    e0 = plsc.cumsum(g0) - g0
    e1 = plsc.cumsum(g1) - g1 + tot0
    offbuf[pl.ds(0, 16)] = e0
    offbuf[pl.ds(16, 16)] = e1
    widv = jnp.full((16,), wid, jnp.int32)
    a = plsc.load_gather(offbuf, [widv])[0]
    c1 = plsc.load_gather(cntbuf, [widv * 16])[0]
    b = a + c1

    sxv, syv, szv = pbuf[0, :], pbuf[1, :], pbuf[2, :]
    oxv, oyv, ozv = pbuf[3, :], pbuf[4, :], pbuf[5, :]

    chunk0 = wid * C
    head = jnp.minimum((8 - (a & 7)) & 7, c1)

    def edge_occ_ring(start, ln):
        @pl.when(ln > 0)
        def _():
            for k in range(8):
                q = jnp.minimum(k * 16 + lanes, ln - 1)
                idxv = start + q
                eidx[pl.ds(k * 16, 16)] = idxv
                evali[pl.ds(k * 16, 16)] = plsc.load_gather(oring, [idxv & RM])
            pltpu.async_copy(evali, occ_hbm.at[eidx], sem_e).wait()

    def edge_xw_ring(start, ln):
        @pl.when(ln > 0)
        def _():
            for k in range(8):
                q = jnp.minimum(k * 16 + lanes, 3 * ln - 1)
                qd = (q * 11) >> 5
                qc = q - qd * 3
                r = start + qd
                eidx[pl.ds(k * 16, 16)] = r * 3 + qc
                evalf[pl.ds(k * 16, 16)] = plsc.load_gather(
                    xring, [(r & RM) * 3 + qc])
            pltpu.async_copy(evalf, xw_hbm.at[eidx], sem_e).wait()

    def edge_occ_zero(start, ln):
        @pl.when(ln > 0)
        def _():
            for k in range(8):
                q = jnp.minimum(k * 16 + lanes, ln - 1)
                eidx[pl.ds(k * 16, 16)] = start + q
                evali[pl.ds(k * 16, 16)] = jnp.zeros((16,), jnp.int32)
            pltpu.async_copy(evali, occ_hbm.at[eidx], sem_e).wait()

    def edge_xw_fill(start, ln):
        @pl.when(ln > 0)
        def _():
            for k in range(8):
                q = jnp.minimum(k * 16 + lanes, 3 * ln - 1)
                qd = (q * 11) >> 5
                qc = q - qd * 3
                eidx[pl.ds(k * 16, 16)] = (start + qd) * 3 + qc
                evalf[pl.ds(k * 16, 16)] = plsc.load_gather(fbufv, [qc])
            pltpu.async_copy(evalf, xw_hbm.at[eidx], sem_e).wait()

    def subchunk(si, carry):
        o1s, f, fired = carry
        base = chunk0 + si * S
        pltpu.sync_copy(mask_hbm.at[pl.ds(_mo8(base), S)], mbuf)
        pltpu.sync_copy(noise_hbm.at[pl.ds(_mo8(base * 3), 3 * S)], nbuf)

        def group(g, o1):
            mv = mbuf[pl.ds(g * 16, 16)]
            m = mv > 0
            incl = plsc.cumsum(mv)
            excl = incl - mv
            t1 = jnp.sum(mv)
            ridx = (o1 + excl) & RM
            lrow = g * 16 + lanes
            vidx = base + lrow
            fi = (vidx >> 14).astype(jnp.float32)
            fj = ((vidx >> 7) & 127).astype(jnp.float32)
            fk = (vidx & 127).astype(jnp.float32)
            nx = plsc.load_gather(nbuf, [lrow * 3])
            ny = plsc.load_gather(nbuf, [lrow * 3 + 1])
            nz = plsc.load_gather(nbuf, [lrow * 3 + 2])
            yx = (fi + nx) * sxv + oxv
            yy = (fj + ny) * syv + oyv
            yz = (fk + nz) * szv + ozv
            r3 = ridx * 3
            plsc.store_scatter(oring, [ridx], vidx, mask=m)
            plsc.store_scatter(xring, [r3], yx, mask=m)
            plsc.store_scatter(xring, [r3 + 1], yy, mask=m)
            plsc.store_scatter(xring, [r3 + 2], yz, mask=m)
            mm = jnp.logical_and(m, ridx < BLK)
            rs3 = (ridx + RING) * 3
            plsc.store_scatter(oring, [ridx + RING], vidx, mask=mm)
            plsc.store_scatter(xring, [rs3], yx, mask=mm)
            plsc.store_scatter(xring, [rs3 + 1], yy, mask=mm)
            plsc.store_scatter(xring, [rs3 + 2], yz, mask=mm)
            return o1 + t1

        o1s = lax.fori_loop(0, GROUPS, group, o1s)

        pend = jnp.maximum(o1s - f, 0)
        nb = pend >> 11

        def flush(j, fcur):
            soff = fcur & RM
            pltpu.sync_copy(oring.at[pl.ds(_mo8(soff), BLK)],
                            occ_hbm.at[pl.ds(_mo8(fcur), BLK)])
            pltpu.sync_copy(xring.at[pl.ds(_mo8(soff * 3), 3 * BLK)],
                            xw_hbm.at[pl.ds(_mo8(fcur * 3), 3 * BLK)])
            return fcur + BLK

        f = lax.fori_loop(0, nb, flush, f)

        fire = jnp.logical_and(fired == 0,
                               jnp.logical_or(o1s - a >= head, si == SUBCH - 1))

        @pl.when(fire)
        def _():
            edge_occ_ring(a, head)
            edge_xw_ring(a, head)

        fired = jnp.where(fire, 1, fired)
        return (o1s, f, fired)

    f0 = a + head
    _, f, _ = lax.fori_loop(0, SUBCH, subchunk,
                            (a, f0, jnp.int32(0)))

    z = C - c1
    zend = total + chunk0 + C - b
    zstart = zend - z
    headf = jnp.minimum((8 - (zstart & 7)) & 7, z)
    ff0 = zstart + headf
    nbf = jnp.maximum(z - headf, 0) >> 11

    def fillblk(j, fcur):
        pltpu.async_copy(zbufv, occ_hbm.at[pl.ds(_mo8(fcur), BLK)], sem_f)
        pltpu.async_copy(fbufv, xw_hbm.at[pl.ds(_mo8(fcur * 3), 3 * BLK)],
                         sem_f)
        return fcur + BLK

    ffb = lax.fori_loop(0, nbf, fillblk, ff0)

    for bit in (1024, 512, 256, 128, 64, 32, 16, 8):
        cond = ((b - f) & bit) != 0

        @pl.when(cond)
        def _(f=f, bit=bit):
            soff = f & RM
            pltpu.sync_copy(oring.at[pl.ds(_mo8(soff), bit)],
                            occ_hbm.at[pl.ds(_mo8(f), bit)])
            pltpu.sync_copy(xring.at[pl.ds(_mo8(soff * 3), 3 * bit)],
                            xw_hbm.at[pl.ds(_mo8(f * 3), 3 * bit)])

        f = jnp.where(cond, f + bit, f)

    edge_occ_ring(f, b - f)

    edge_xw_ring(f, b - f)

    ffc = ffb
    for bit in (1024, 512, 256, 128, 64, 32, 16, 8):
        cond = ((zend - ffc) & bit) != 0

        @pl.when(cond)
        def _(ffc=ffc, bit=bit):
            pltpu.sync_copy(zbufv.at[pl.ds(0, bit)],
                            occ_hbm.at[pl.ds(_mo8(ffc), bit)])
            pltpu.sync_copy(fbufv.at[pl.ds(0, 3 * bit)],
                            xw_hbm.at[pl.ds(_mo8(ffc * 3), 3 * bit)])

        ffc = jnp.where(cond, ffc + bit, ffc)

    edge_occ_zero(ffc, zend - ffc)
    edge_xw_fill(ffc, zend - ffc)
    edge_occ_zero(zstart, headf)
    edge_xw_fill(zstart, headf)

    def filldrain(j, fcur):
        pltpu.make_async_copy(
            zbufv, occ_hbm.at[pl.ds(_mo8(fcur), BLK)], sem_f).wait()
        pltpu.make_async_copy(
            fbufv, xw_hbm.at[pl.ds(_mo8(fcur * 3), 3 * BLK)], sem_f).wait()
        return fcur + BLK

    lax.fori_loop(0, nbf, filldrain, ff0)


def kernel(binary, noise, roi_aabb):
    mask = binary.reshape(-1).astype(jnp.int32)
    noise_flat = noise.reshape(-1)
    scale = (roi_aabb[3:] - roi_aabb[:3]) / jnp.float32(RES)
    offset = roi_aabb[:3]
    yfill = noise[0] * scale + offset
    params = jnp.tile(
        jnp.concatenate([scale, offset, yfill])[:, None], (1, 16))
    fpat = jnp.tile(yfill, BLK)
    zpat = jnp.zeros((BLK,), jnp.int32)

    mesh = plsc.VectorSubcoreMesh(
        core_axis_name="c", subcore_axis_name="s",
        num_cores=NC, num_subcores=NS)

    count_k = pl.kernel(
        _count_body,
        out_type=jax.ShapeDtypeStruct((NW * 16,), jnp.int32),
        mesh=mesh,
        compiler_params=_CPARAMS,
        scratch_types=[
            pltpu.VMEM((CNT_S,), jnp.int32),
            pltpu.VMEM((16,), jnp.int32),
        ])
    counts = count_k(mask)

    main_k = pl.kernel(
        _main_body,
        out_type=(jax.ShapeDtypeStruct((3 * N,), jnp.float32),
                  jax.ShapeDtypeStruct((N,), jnp.int32)),
        mesh=mesh,
        compiler_params=_CPARAMS,
        scratch_types=[
            pltpu.VMEM((S,), jnp.int32),
            pltpu.VMEM((3 * S,), jnp.float32),
            pltpu.VMEM((RING + BLK,), jnp.int32),
            pltpu.VMEM(((RING + BLK) * 3,), jnp.float32),
            pltpu.VMEM((9, 16), jnp.float32),
            pltpu.VMEM((NW * 16,), jnp.int32),
            pltpu.VMEM((32,), jnp.int32),
            pltpu.VMEM((BLK,), jnp.int32),
            pltpu.VMEM((3 * BLK,), jnp.float32),
            pltpu.VMEM((128,), jnp.int32),
            pltpu.VMEM((128,), jnp.int32),
            pltpu.VMEM((128,), jnp.float32),
            pltpu.SemaphoreType.DMA,
            pltpu.SemaphoreType.DMA,
        ])
    xw_flat, occ_indices = main_k(mask, noise_flat, params, counts,
                                  fpat, zpat)
    return xw_flat.reshape(N, 3), occ_indices


# speedup vs baseline: 4.5468x; 1.0011x over previous
import jax
import jax.numpy as jnp
from jax import lax
from jax.experimental import pallas as pl
from jax.experimental.pallas import tpu as pltpu
from jax.experimental.pallas import tpu_sc as plsc

RES = 128
N = RES ** 3
NC, NS = 2, 16
NW = NC * NS
C = N // NW
S = 4096
GROUPS = S // 16
SUBCH = C // S
CNT_S = 8192
RING = 8192
RM = RING - 1
BLK = 2048

_CPARAMS = pltpu.CompilerParams(
    needs_layout_passes=False, disable_bounds_checks=True)


def _mo8(x):
    return pl.multiple_of(x, 8)


def _wid():
    return lax.axis_index("s") * NC + lax.axis_index("c")


def _count_body(mask_hbm, counts_hbm, mbuf, cbuf):
    wid = _wid()
    acc = jnp.zeros((16,), jnp.int32)
    for si in range(C // CNT_S):
        base = pl.multiple_of(wid * C + si * CNT_S, CNT_S)
        pltpu.sync_copy(mask_hbm.at[pl.ds(base, CNT_S)], mbuf)
        acc = lax.fori_loop(
            0, CNT_S // 16,
            lambda g, a: a + mbuf[pl.ds(g * 16, 16)],
            acc)
    cbuf[...] = jnp.full((16,), jnp.sum(acc), jnp.int32)
    pltpu.sync_copy(cbuf, counts_hbm.at[pl.ds(pl.multiple_of(wid * 16, 16), 16)])


def _main_body(mask_hbm, noise_hbm, params_hbm, counts_hbm, fpat_hbm, zpat_hbm,
               xw_hbm, occ_hbm,
               mbuf, nbuf, oring, xring, pbuf, cntbuf, offbuf,
               zbufv, fbufv, eidx, evali, evalf,
               sem_e, sem_f):
    wid = _wid()
    lanes = lax.iota(jnp.int32, 16)

    pltpu.sync_copy(params_hbm, pbuf)
    pltpu.sync_copy(counts_hbm, cntbuf)
    pltpu.sync_copy(zpat_hbm, zbufv)
    pltpu.sync_copy(fpat_hbm, fbufv)

    g0 = plsc.load_gather(cntbuf, [lanes * 16])
    g1 = plsc.load_gather(cntbuf, [256 + lanes * 16])
    tot0 = jnp.sum(g0)
    total = tot0 + jnp.sum(g1)
    e0 = plsc.cumsum(g0) - g0
    e1 = plsc.cumsum(g1) - g1 + tot0
    offbuf[pl.ds(0, 16)] = e0
    offbuf[pl.ds(16, 16)] = e1
    widv = jnp.full((16,), wid, jnp.int32)
    a = plsc.load_gather(offbuf, [widv])[0]
    c1 = plsc.load_gather(cntbuf, [widv * 16])[0]
    b = a + c1

    sxv, syv, szv = pbuf[0, :], pbuf[1, :], pbuf[2, :]
    oxv, oyv, ozv = pbuf[3, :], pbuf[4, :], pbuf[5, :]

    chunk0 = wid * C
    head = jnp.minimum((8 - (a & 7)) & 7, c1)

    def edge_occ_ring(start, ln):
        @pl.when(ln > 0)
        def _():
            for k in range(8):
                q = jnp.minimum(k * 16 + lanes, ln - 1)
                idxv = start + q
                eidx[pl.ds(k * 16, 16)] = idxv
                evali[pl.ds(k * 16, 16)] = plsc.load_gather(oring, [idxv & RM])
            pltpu.async_copy(evali, occ_hbm.at[eidx], sem_e).wait()

    def edge_xw_ring(start, ln):
        @pl.when(ln > 0)
        def _():
            for k in range(8):
                q = jnp.minimum(k * 16 + lanes, 3 * ln - 1)
                qd = (q * 11) >> 5
                qc = q - qd * 3
                r = start + qd
                eidx[pl.ds(k * 16, 16)] = r * 3 + qc
                evalf[pl.ds(k * 16, 16)] = plsc.load_gather(
                    xring, [(r & RM) * 3 + qc])
            pltpu.async_copy(evalf, xw_hbm.at[eidx], sem_e).wait()

    def edge_occ_zero(start, ln):
        @pl.when(ln > 0)
        def _():
            for k in range(8):
                q = jnp.minimum(k * 16 + lanes, ln - 1)
                eidx[pl.ds(k * 16, 16)] = start + q
                evali[pl.ds(k * 16, 16)] = jnp.zeros((16,), jnp.int32)
            pltpu.async_copy(evali, occ_hbm.at[eidx], sem_e).wait()

    def edge_xw_fill(start, ln):
        @pl.when(ln > 0)
        def _():
            for k in range(8):
                q = jnp.minimum(k * 16 + lanes, 3 * ln - 1)
                qd = (q * 11) >> 5
                qc = q - qd * 3
                eidx[pl.ds(k * 16, 16)] = (start + qd) * 3 + qc
                evalf[pl.ds(k * 16, 16)] = plsc.load_gather(fbufv, [qc])
            pltpu.async_copy(evalf, xw_hbm.at[eidx], sem_e).wait()

    def subchunk(si, carry):
        o1s, f, fired = carry
        base = chunk0 + si * S
        pltpu.sync_copy(mask_hbm.at[pl.ds(_mo8(base), S)], mbuf)
        pltpu.sync_copy(noise_hbm.at[pl.ds(_mo8(base * 3), 3 * S)], nbuf)

        def one_group(g, o1):
            mv = mbuf[pl.ds(g * 16, 16)]
            m = mv > 0
            incl = plsc.cumsum(mv)
            excl = incl - mv
            t1 = jnp.sum(mv)
            ridx = (o1 + excl) & RM
            lrow = g * 16 + lanes
            vidx = base + lrow
            fi = (vidx >> 14).astype(jnp.float32)
            fj = ((vidx >> 7) & 127).astype(jnp.float32)
            fk = (vidx & 127).astype(jnp.float32)
            nx = plsc.load_gather(nbuf, [lrow * 3])
            ny = plsc.load_gather(nbuf, [lrow * 3 + 1])
            nz = plsc.load_gather(nbuf, [lrow * 3 + 2])
            yx = (fi + nx) * sxv + oxv
            yy = (fj + ny) * syv + oyv
            yz = (fk + nz) * szv + ozv
            r3 = ridx * 3
            plsc.store_scatter(oring, [ridx], vidx, mask=m)
            plsc.store_scatter(xring, [r3], yx, mask=m)
            plsc.store_scatter(xring, [r3 + 1], yy, mask=m)
            plsc.store_scatter(xring, [r3 + 2], yz, mask=m)
            mm = jnp.logical_and(m, ridx < BLK)
            rs3 = (ridx + RING) * 3
            plsc.store_scatter(oring, [ridx + RING], vidx, mask=mm)
            plsc.store_scatter(xring, [rs3], yx, mask=mm)
            plsc.store_scatter(xring, [rs3 + 1], yy, mask=mm)
            plsc.store_scatter(xring, [rs3 + 2], yz, mask=mm)
            return o1 + t1

        def group4(gq, o1):
            for u in range(4):
                o1 = one_group(gq * 4 + u, o1)
            return o1

        o1s = lax.fori_loop(0, GROUPS // 4, group4, o1s)

        pend = jnp.maximum(o1s - f, 0)
        nb = pend >> 11

        def flush(j, fcur):
            soff = fcur & RM
            pltpu.sync_copy(oring.at[pl.ds(_mo8(soff), BLK)],
                            occ_hbm.at[pl.ds(_mo8(fcur), BLK)])
            pltpu.sync_copy(xring.at[pl.ds(_mo8(soff * 3), 3 * BLK)],
                            xw_hbm.at[pl.ds(_mo8(fcur * 3), 3 * BLK)])
            return fcur + BLK

        f = lax.fori_loop(0, nb, flush, f)

        fire = jnp.logical_and(fired == 0,
                               jnp.logical_or(o1s - a >= head, si == SUBCH - 1))

        @pl.when(fire)
        def _():
            edge_occ_ring(a, head)
            edge_xw_ring(a, head)

        fired = jnp.where(fire, 1, fired)
        return (o1s, f, fired)

    f0 = a + head
    _, f, _ = lax.fori_loop(0, SUBCH, subchunk,
                            (a, f0, jnp.int32(0)))

    z = C - c1
    zend = total + chunk0 + C - b
    zstart = zend - z
    headf = jnp.minimum((8 - (zstart & 7)) & 7, z)
    ff0 = zstart + headf
    nbf = jnp.maximum(z - headf, 0) >> 11

    def fillblk(j, fcur):
        pltpu.async_copy(zbufv, occ_hbm.at[pl.ds(_mo8(fcur), BLK)], sem_f)
        pltpu.async_copy(fbufv, xw_hbm.at[pl.ds(_mo8(fcur * 3), 3 * BLK)],
                         sem_f)
        return fcur + BLK

    ffb = lax.fori_loop(0, nbf, fillblk, ff0)

    for bit in (1024, 512, 256, 128, 64, 32, 16, 8):
        cond = ((b - f) & bit) != 0

        @pl.when(cond)
        def _(f=f, bit=bit):
            soff = f & RM
            pltpu.sync_copy(oring.at[pl.ds(_mo8(soff), bit)],
                            occ_hbm.at[pl.ds(_mo8(f), bit)])
            pltpu.sync_copy(xring.at[pl.ds(_mo8(soff * 3), 3 * bit)],
                            xw_hbm.at[pl.ds(_mo8(f * 3), 3 * bit)])

        f = jnp.where(cond, f + bit, f)

    edge_occ_ring(f, b - f)

    edge_xw_ring(f, b - f)

    ffc = ffb
    for bit in (1024, 512, 256, 128, 64, 32, 16, 8):
        cond = ((zend - ffc) & bit) != 0

        @pl.when(cond)
        def _(ffc=ffc, bit=bit):
            pltpu.sync_copy(zbufv.at[pl.ds(0, bit)],
                            occ_hbm.at[pl.ds(_mo8(ffc), bit)])
            pltpu.sync_copy(fbufv.at[pl.ds(0, 3 * bit)],
                            xw_hbm.at[pl.ds(_mo8(ffc * 3), 3 * bit)])

        ffc = jnp.where(cond, ffc + bit, ffc)

    edge_occ_zero(ffc, zend - ffc)
    edge_xw_fill(ffc, zend - ffc)
    edge_occ_zero(zstart, headf)
    edge_xw_fill(zstart, headf)

    def filldrain(j, fcur):
        pltpu.make_async_copy(
            zbufv, occ_hbm.at[pl.ds(_mo8(fcur), BLK)], sem_f).wait()
        pltpu.make_async_copy(
            fbufv, xw_hbm.at[pl.ds(_mo8(fcur * 3), 3 * BLK)], sem_f).wait()
        return fcur + BLK

    lax.fori_loop(0, nbf, filldrain, ff0)


def kernel(binary, noise, roi_aabb):
    mask = binary.reshape(-1).astype(jnp.int32)
    noise_flat = noise.reshape(-1)
    scale = (roi_aabb[3:] - roi_aabb[:3]) / jnp.float32(RES)
    offset = roi_aabb[:3]
    yfill = noise[0] * scale + offset
    params = jnp.tile(
        jnp.concatenate([scale, offset, yfill])[:, None], (1, 16))
    fpat = jnp.tile(yfill, BLK)
    zpat = jnp.zeros((BLK,), jnp.int32)

    mesh = plsc.VectorSubcoreMesh(
        core_axis_name="c", subcore_axis_name="s",
        num_cores=NC, num_subcores=NS)

    count_k = pl.kernel(
        _count_body,
        out_type=jax.ShapeDtypeStruct((NW * 16,), jnp.int32),
        mesh=mesh,
        compiler_params=_CPARAMS,
        scratch_types=[
            pltpu.VMEM((CNT_S,), jnp.int32),
            pltpu.VMEM((16,), jnp.int32),
        ])
    counts = count_k(mask)

    main_k = pl.kernel(
        _main_body,
        out_type=(jax.ShapeDtypeStruct((3 * N,), jnp.float32),
                  jax.ShapeDtypeStruct((N,), jnp.int32)),
        mesh=mesh,
        compiler_params=_CPARAMS,
        scratch_types=[
            pltpu.VMEM((S,), jnp.int32),
            pltpu.VMEM((3 * S,), jnp.float32),
            pltpu.VMEM((RING + BLK,), jnp.int32),
            pltpu.VMEM(((RING + BLK) * 3,), jnp.float32),
            pltpu.VMEM((9, 16), jnp.float32),
            pltpu.VMEM((NW * 16,), jnp.int32),
            pltpu.VMEM((32,), jnp.int32),
            pltpu.VMEM((BLK,), jnp.int32),
            pltpu.VMEM((3 * BLK,), jnp.float32),
            pltpu.VMEM((128,), jnp.int32),
            pltpu.VMEM((128,), jnp.int32),
            pltpu.VMEM((128,), jnp.float32),
            pltpu.SemaphoreType.DMA,
            pltpu.SemaphoreType.DMA,
        ])
    xw_flat, occ_indices = main_k(mask, noise_flat, params, counts,
                                  fpat, zpat)
    return xw_flat.reshape(N, 3), occ_indices


# speedup vs baseline: 4.5478x; 1.0002x over previous
import jax
import jax.numpy as jnp
from jax import lax
from jax.experimental import pallas as pl
from jax.experimental.pallas import tpu as pltpu
from jax.experimental.pallas import tpu_sc as plsc

RES = 128
N = RES ** 3
NC, NS = 2, 16
NW = NC * NS
C = N // NW
S = 4096
GROUPS = S // 16
SUBCH = C // S
CNT_S = 8192
RING = 8192
RM = RING - 1
BLK = 2048

_CPARAMS = pltpu.CompilerParams(
    needs_layout_passes=False, disable_bounds_checks=True)


def _mo8(x):
    return pl.multiple_of(x, 8)


def _wid():
    return lax.axis_index("s") * NC + lax.axis_index("c")


def _count_body(mask_hbm, counts_hbm, mbuf, cbuf):
    wid = _wid()
    acc = jnp.zeros((16,), jnp.int32)
    for si in range(C // CNT_S):
        base = pl.multiple_of(wid * C + si * CNT_S, CNT_S)
        pltpu.sync_copy(mask_hbm.at[pl.ds(base, CNT_S)], mbuf)
        acc = lax.fori_loop(
            0, CNT_S // 16,
            lambda g, a: a + mbuf[pl.ds(g * 16, 16)],
            acc)
    cbuf[...] = jnp.full((16,), jnp.sum(acc), jnp.int32)
    pltpu.sync_copy(cbuf, counts_hbm.at[pl.ds(pl.multiple_of(wid * 16, 16), 16)])


def _main_body(mask_hbm, noise_hbm, params_hbm, counts_hbm, fpat_hbm, zpat_hbm,
               xw_hbm, occ_hbm,
               mbuf, nbuf, oring, xring, pbuf, cntbuf, offbuf,
               zbufv, fbufv, eidx, evali, evalf,
               sem_e, sem_f):
    wid = _wid()
    lanes = lax.iota(jnp.int32, 16)

    pltpu.sync_copy(params_hbm, pbuf)
    pltpu.sync_copy(counts_hbm, cntbuf)
    pltpu.sync_copy(zpat_hbm, zbufv)
    pltpu.sync_copy(fpat_hbm, fbufv)

    g0 = plsc.load_gather(cntbuf, [lanes * 16])
    g1 = plsc.load_gather(cntbuf, [256 + lanes * 16])
    tot0 = jnp.sum(g0)
    total = tot0 + jnp.sum(g1)
    e0 = plsc.cumsum(g0) - g0
    e1 = plsc.cumsum(g1) - g1 + tot0
    offbuf[pl.ds(0, 16)] = e0
    offbuf[pl.ds(16, 16)] = e1
    widv = jnp.full((16,), wid, jnp.int32)
    a = plsc.load_gather(offbuf, [widv])[0]
    c1 = plsc.load_gather(cntbuf, [widv * 16])[0]
    b = a + c1

    sxv, syv, szv = pbuf[0, :], pbuf[1, :], pbuf[2, :]
    oxv, oyv, ozv = pbuf[3, :], pbuf[4, :], pbuf[5, :]

    chunk0 = wid * C
    head = jnp.minimum((8 - (a & 7)) & 7, c1)

    def refresh_shadow():
        def cp(i, _):
            oring[pl.ds(RING + i * 16, 16)] = oring[pl.ds(i * 16, 16)]
            return 0
        lax.fori_loop(0, BLK // 16, cp, 0)

        def cpx(i, _):
            xring[pl.ds(3 * RING + i * 16, 16)] = xring[pl.ds(i * 16, 16)]
            return 0
        lax.fori_loop(0, 3 * BLK // 16, cpx, 0)

    def edge_occ_ring(start, ln):
        @pl.when(ln > 0)
        def _():
            for k in range(8):
                q = jnp.minimum(k * 16 + lanes, ln - 1)
                idxv = start + q
                eidx[pl.ds(k * 16, 16)] = idxv
                evali[pl.ds(k * 16, 16)] = plsc.load_gather(oring, [idxv & RM])
            pltpu.async_copy(evali, occ_hbm.at[eidx], sem_e).wait()

    def edge_xw_ring(start, ln):
        @pl.when(ln > 0)
        def _():
            for k in range(8):
                q = jnp.minimum(k * 16 + lanes, 3 * ln - 1)
                qd = (q * 11) >> 5
                qc = q - qd * 3
                r = start + qd
                eidx[pl.ds(k * 16, 16)] = r * 3 + qc
                evalf[pl.ds(k * 16, 16)] = plsc.load_gather(
                    xring, [(r & RM) * 3 + qc])
            pltpu.async_copy(evalf, xw_hbm.at[eidx], sem_e).wait()

    def edge_occ_zero(start, ln):
        @pl.when(ln > 0)
        def _():
            for k in range(8):
                q = jnp.minimum(k * 16 + lanes, ln - 1)
                eidx[pl.ds(k * 16, 16)] = start + q
                evali[pl.ds(k * 16, 16)] = jnp.zeros((16,), jnp.int32)
            pltpu.async_copy(evali, occ_hbm.at[eidx], sem_e).wait()

    def edge_xw_fill(start, ln):
        @pl.when(ln > 0)
        def _():
            for k in range(8):
                q = jnp.minimum(k * 16 + lanes, 3 * ln - 1)
                qd = (q * 11) >> 5
                qc = q - qd * 3
                eidx[pl.ds(k * 16, 16)] = (start + qd) * 3 + qc
                evalf[pl.ds(k * 16, 16)] = plsc.load_gather(fbufv, [qc])
            pltpu.async_copy(evalf, xw_hbm.at[eidx], sem_e).wait()

    def subchunk(si, carry):
        o1s, f, fired = carry
        base = chunk0 + si * S
        pltpu.sync_copy(mask_hbm.at[pl.ds(_mo8(base), S)], mbuf)
        pltpu.sync_copy(noise_hbm.at[pl.ds(_mo8(base * 3), 3 * S)], nbuf)

        def one_group(g, o1):
            mv = mbuf[pl.ds(g * 16, 16)]
            m = mv > 0
            incl = plsc.cumsum(mv)
            excl = incl - mv
            t1 = jnp.sum(mv)
            ridx = (o1 + excl) & RM
            lrow = g * 16 + lanes
            vidx = base + lrow
            fi = (vidx >> 14).astype(jnp.float32)
            fj = ((vidx >> 7) & 127).astype(jnp.float32)
            fk = (vidx & 127).astype(jnp.float32)
            nx = plsc.load_gather(nbuf, [lrow * 3])
            ny = plsc.load_gather(nbuf, [lrow * 3 + 1])
            nz = plsc.load_gather(nbuf, [lrow * 3 + 2])
            yx = (fi + nx) * sxv + oxv
            yy = (fj + ny) * syv + oyv
            yz = (fk + nz) * szv + ozv
            r3 = ridx * 3
            plsc.store_scatter(oring, [ridx], vidx, mask=m)
            plsc.store_scatter(xring, [r3], yx, mask=m)
            plsc.store_scatter(xring, [r3 + 1], yy, mask=m)
            plsc.store_scatter(xring, [r3 + 2], yz, mask=m)
            return o1 + t1

        def group4(gq, o1):
            for u in range(4):
                o1 = one_group(gq * 4 + u, o1)
            return o1

        o1s = lax.fori_loop(0, GROUPS // 4, group4, o1s)

        pend = jnp.maximum(o1s - f, 0)
        nb = pend >> 11

        def flush(j, fcur):
            soff = fcur & RM

            @pl.when(soff > RING - BLK)
            def _():
                refresh_shadow()

            pltpu.sync_copy(oring.at[pl.ds(_mo8(soff), BLK)],
                            occ_hbm.at[pl.ds(_mo8(fcur), BLK)])
            pltpu.sync_copy(xring.at[pl.ds(_mo8(soff * 3), 3 * BLK)],
                            xw_hbm.at[pl.ds(_mo8(fcur * 3), 3 * BLK)])
            return fcur + BLK

        f = lax.fori_loop(0, nb, flush, f)

        fire = jnp.logical_and(fired == 0,
                               jnp.logical_or(o1s - a >= head, si == SUBCH - 1))

        @pl.when(fire)
        def _():
            edge_occ_ring(a, head)
            edge_xw_ring(a, head)

        fired = jnp.where(fire, 1, fired)
        return (o1s, f, fired)

    f0 = a + head
    _, f, _ = lax.fori_loop(0, SUBCH, subchunk,
                            (a, f0, jnp.int32(0)))

    z = C - c1
    zend = total + chunk0 + C - b
    zstart = zend - z
    headf = jnp.minimum((8 - (zstart & 7)) & 7, z)
    ff0 = zstart + headf
    nbf = jnp.maximum(z - headf, 0) >> 11

    def fillblk(j, fcur):
        pltpu.async_copy(zbufv, occ_hbm.at[pl.ds(_mo8(fcur), BLK)], sem_f)
        pltpu.async_copy(fbufv, xw_hbm.at[pl.ds(_mo8(fcur * 3), 3 * BLK)],
                         sem_f)
        return fcur + BLK

    ffb = lax.fori_loop(0, nbf, fillblk, ff0)

    refresh_shadow()
    for bit in (1024, 512, 256, 128, 64, 32, 16, 8):
        cond = ((b - f) & bit) != 0

        @pl.when(cond)
        def _(f=f, bit=bit):
            soff = f & RM
            pltpu.sync_copy(oring.at[pl.ds(_mo8(soff), bit)],
                            occ_hbm.at[pl.ds(_mo8(f), bit)])
            pltpu.sync_copy(xring.at[pl.ds(_mo8(soff * 3), 3 * bit)],
                            xw_hbm.at[pl.ds(_mo8(f * 3), 3 * bit)])

        f = jnp.where(cond, f + bit, f)

    edge_occ_ring(f, b - f)

    edge_xw_ring(f, b - f)

    ffc = ffb
    for bit in (1024, 512, 256, 128, 64, 32, 16, 8):
        cond = ((zend - ffc) & bit) != 0

        @pl.when(cond)
        def _(ffc=ffc, bit=bit):
            pltpu.sync_copy(zbufv.at[pl.ds(0, bit)],
                            occ_hbm.at[pl.ds(_mo8(ffc), bit)])
            pltpu.sync_copy(fbufv.at[pl.ds(0, 3 * bit)],
                            xw_hbm.at[pl.ds(_mo8(ffc * 3), 3 * bit)])

        ffc = jnp.where(cond, ffc + bit, ffc)

    edge_occ_zero(ffc, zend - ffc)
    edge_xw_fill(ffc, zend - ffc)
    edge_occ_zero(zstart, headf)
    edge_xw_fill(zstart, headf)

    def filldrain(j, fcur):
        pltpu.make_async_copy(
            zbufv, occ_hbm.at[pl.ds(_mo8(fcur), BLK)], sem_f).wait()
        pltpu.make_async_copy(
            fbufv, xw_hbm.at[pl.ds(_mo8(fcur * 3), 3 * BLK)], sem_f).wait()
        return fcur + BLK

    lax.fori_loop(0, nbf, filldrain, ff0)


def kernel(binary, noise, roi_aabb):
    mask = binary.reshape(-1).astype(jnp.int32)
    noise_flat = noise.reshape(-1)
    scale = (roi_aabb[3:] - roi_aabb[:3]) / jnp.float32(RES)
    offset = roi_aabb[:3]
    yfill = noise[0] * scale + offset
    params = jnp.tile(
        jnp.concatenate([scale, offset, yfill])[:, None], (1, 16))
    fpat = jnp.tile(yfill, BLK)
    zpat = jnp.zeros((BLK,), jnp.int32)

    mesh = plsc.VectorSubcoreMesh(
        core_axis_name="c", subcore_axis_name="s",
        num_cores=NC, num_subcores=NS)

    count_k = pl.kernel(
        _count_body,
        out_type=jax.ShapeDtypeStruct((NW * 16,), jnp.int32),
        mesh=mesh,
        compiler_params=_CPARAMS,
        scratch_types=[
            pltpu.VMEM((CNT_S,), jnp.int32),
            pltpu.VMEM((16,), jnp.int32),
        ])
    counts = count_k(mask)

    main_k = pl.kernel(
        _main_body,
        out_type=(jax.ShapeDtypeStruct((3 * N,), jnp.float32),
                  jax.ShapeDtypeStruct((N,), jnp.int32)),
        mesh=mesh,
        compiler_params=_CPARAMS,
        scratch_types=[
            pltpu.VMEM((S,), jnp.int32),
            pltpu.VMEM((3 * S,), jnp.float32),
            pltpu.VMEM((RING + BLK,), jnp.int32),
            pltpu.VMEM(((RING + BLK) * 3,), jnp.float32),
            pltpu.VMEM((9, 16), jnp.float32),
            pltpu.VMEM((NW * 16,), jnp.int32),
            pltpu.VMEM((32,), jnp.int32),
            pltpu.VMEM((BLK,), jnp.int32),
            pltpu.VMEM((3 * BLK,), jnp.float32),
            pltpu.VMEM((128,), jnp.int32),
            pltpu.VMEM((128,), jnp.int32),
            pltpu.VMEM((128,), jnp.float32),
            pltpu.SemaphoreType.DMA,
            pltpu.SemaphoreType.DMA,
        ])
    xw_flat, occ_indices = main_k(mask, noise_flat, params, counts,
                                  fpat, zpat)
    return xw_flat.reshape(N, 3), occ_indices
